# speedup vs baseline: 1.0034x; 1.0034x over previous
import functools
import math

import jax
import jax.numpy as jnp
from jax import lax
from jax.experimental import pallas as pl
from jax.experimental.pallas import tpu as pltpu


def _sink_body(eps_ref, rows_ref, xyt_ref, out_ref, pots_ref, *, R, NB, nlog):
    p = pl.program_id(0)
    t = pl.program_id(1)
    s = pl.program_id(2)
    i = pl.program_id(3)
    T = pl.num_programs(1)
    N = NB * R

    eps = eps_ref[t]
    inv_eps = 1.0 / eps
    col_sel = jnp.where(p == 0, 1 - s, s)
    rb = lax.rem(t, 2)
    wb = 1 - rb

    rows = rows_ref[0]
    rsq = jnp.sum(rows * rows, axis=1, keepdims=True)

    is_first = t == 0
    mid = jnp.logical_and(t >= 1, t <= T - 2)
    neg_half_inv_eps = -0.5 * inv_eps

    m = jnp.full((R, 1), -jnp.inf, jnp.float32)
    ssum = jnp.zeros((R, 1), jnp.float32)
    for c in range(NB):
        colsk = xyt_ref[col_sel, :, c * R:(c + 1) * R]
        csqk = jnp.sum(colsk * colsk, axis=0, keepdims=True)
        dk = jnp.dot(rows, colsk, preferred_element_type=jnp.float32)
        sq = rsq + (csqk - 2.0 * dk)
        phk = pots_ref[col_sel, rb, c]
        hk = jnp.where(is_first, nlog, nlog + phk * inv_eps)
        sc = hk + jnp.maximum(sq, 0.0) * neg_half_inv_eps
        mc = jnp.max(sc, axis=1, keepdims=True)
        mnew = jnp.maximum(m, mc)
        ssum = ssum * jnp.exp(m - mnew) + jnp.sum(
            jnp.exp(sc - mnew), axis=1, keepdims=True)
        m = mnew

    smin = (-eps) * (m + jnp.log(ssum))
    smin_row = smin.T
    old = pots_ref[s, rb, i]
    new = jnp.where(mid, 0.5 * (old + smin_row), smin_row)
    pots_ref[s, wb, i] = new

    @pl.when(t == T - 1)
    def _():
        contrib = jnp.sum(new, axis=1, keepdims=True) * (1.0 / N)
        c3 = contrib.reshape(1, 1, 1)
        first = jnp.logical_and(s == 0, i == 0)
        out_ref[...] = jnp.where(first, c3, out_ref[...] + c3)


def kernel(g, Y, eps_list):
    N, D = g.shape
    M = Y.shape[0]
    assert N == M, "kernel assumes equal-sized point clouds"
    R = 256 if N % 256 == 0 else 128
    NB = N // R
    nlog = -math.log(float(N))

    xy = jnp.stack([g, Y])
    xyt = jnp.stack([g.T, Y.T])
    eps_sched = jnp.concatenate([eps_list[:1], eps_list, eps_list[-1:]])
    T = eps_sched.shape[0]

    body = functools.partial(_sink_body, R=R, NB=NB, nlog=nlog)
    out = pl.pallas_call(
        body,
        grid=(2, T, 2, NB),
        in_specs=[
            pl.BlockSpec(memory_space=pltpu.SMEM),
            pl.BlockSpec((1, R, D), lambda p, t, s, i: (s, i, 0)),
            pl.BlockSpec((2, D, N), lambda p, t, s, i: (0, 0, 0)),
        ],
        out_specs=pl.BlockSpec((1, 1, 1), lambda p, t, s, i: (p, 0, 0)),
        out_shape=jax.ShapeDtypeStruct((2, 1, 1), jnp.float32),
        scratch_shapes=[pltpu.VMEM((2, 2, NB, 1, R), jnp.float32)],
        compiler_params=pltpu.CompilerParams(
            dimension_semantics=("parallel", "arbitrary", "arbitrary",
                                 "arbitrary"),
            vmem_limit_bytes=48 * 1024 * 1024,
        ),
        name="sinkhorn_fused",
    )(eps_sched, xy, xyt)
    return out[0, 0, 0] - out[1, 0, 0]


# speedup vs baseline: 1.1494x; 1.1455x over previous
import functools
import math

import jax
import jax.numpy as jnp
from jax import lax
from jax.experimental import pallas as pl
from jax.experimental.pallas import tpu as pltpu

_LOG2E = 1.4426950408889634
_LN2 = 0.6931471805599453


def _sink_body(eps_ref, xy_ref, xyt_ref, out_ref, pots_ref, bc_ref, *,
               R, NB, nlog):
    p = pl.program_id(0)
    t = pl.program_id(1)
    s = pl.program_id(2)
    T = pl.num_programs(1)
    N = NB * R

    eps = eps_ref[t]
    inv_eps = 1.0 / eps
    alpha2 = (-0.5 * _LOG2E) * inv_eps
    g2 = _LOG2E * inv_eps
    col_sel = jnp.where(p == 0, 1 - s, s)
    rb = lax.rem(t, 2)
    wb = 1 - rb

    is_first = t == 0
    mid = jnp.logical_and(t >= 1, t <= T - 2)
    hbase = nlog * _LOG2E

    for c in range(NB):
        colsk = xyt_ref[col_sel, :, c * R:(c + 1) * R]
        bc_ref[:, c * R:(c + 1) * R] = alpha2 * jnp.sum(
            colsk * colsk, axis=0, keepdims=True)

    def row_block(i, carry):
        rows = xy_ref[s, pl.ds(i * R, R), :]
        br2 = alpha2 * jnp.sum(rows * rows, axis=1, keepdims=True)

        m2 = jnp.full((R, 1), -jnp.inf, jnp.float32)
        s2 = jnp.zeros((R, 1), jnp.float32)
        for c in range(NB):
            colsk = xyt_ref[col_sel, :, c * R:(c + 1) * R]
            dk = jnp.dot(rows, colsk, preferred_element_type=jnp.float32)
            w = jnp.minimum(dk * g2 + br2 + bc_ref[:, c * R:(c + 1) * R], 0.0)
            phk = pots_ref[col_sel, rb, c]
            hk2 = jnp.where(is_first, hbase, hbase + g2 * phk)
            sc2 = w + hk2
            mc = jnp.max(sc2, axis=1, keepdims=True)
            mnew = jnp.maximum(m2, mc)
            s2 = s2 * jnp.exp2(m2 - mnew) + jnp.sum(
                jnp.exp2(sc2 - mnew), axis=1, keepdims=True)
            m2 = mnew

        smin = (-_LN2 * eps) * (m2 + jnp.log2(s2))
        smin_row = smin.T
        old = pots_ref[s, rb, i]
        new = jnp.where(mid, 0.5 * (old + smin_row), smin_row)
        pots_ref[s, wb, i] = new

        @pl.when(t == T - 1)
        def _():
            contrib = jnp.sum(new, axis=1, keepdims=True) * (1.0 / N)
            c3 = contrib.reshape(1, 1, 1)
            first = jnp.logical_and(s == 0, i == 0)
            out_ref[...] = jnp.where(first, c3, out_ref[...] + c3)

        return carry

    lax.fori_loop(0, NB, row_block, 0)


def kernel(g, Y, eps_list):
    N, D = g.shape
    M = Y.shape[0]
    assert N == M, "kernel assumes equal-sized point clouds"
    R = 256 if N % 256 == 0 else 128
    NB = N // R
    nlog = -math.log(float(N))

    xy = jnp.stack([g, Y])
    xyt = jnp.stack([g.T, Y.T])
    eps_sched = jnp.concatenate([eps_list[:1], eps_list, eps_list[-1:]])
    T = eps_sched.shape[0]

    body = functools.partial(_sink_body, R=R, NB=NB, nlog=nlog)
    out = pl.pallas_call(
        body,
        grid=(2, T, 2),
        in_specs=[
            pl.BlockSpec(memory_space=pltpu.SMEM),
            pl.BlockSpec((2, N, D), lambda p, t, s: (0, 0, 0)),
            pl.BlockSpec((2, D, N), lambda p, t, s: (0, 0, 0)),
        ],
        out_specs=pl.BlockSpec((1, 1, 1), lambda p, t, s: (p, 0, 0)),
        out_shape=jax.ShapeDtypeStruct((2, 1, 1), jnp.float32),
        scratch_shapes=[
            pltpu.VMEM((2, 2, NB, 1, R), jnp.float32),
            pltpu.VMEM((1, N), jnp.float32),
        ],
        compiler_params=pltpu.CompilerParams(
            dimension_semantics=("arbitrary", "arbitrary", "arbitrary"),
            vmem_limit_bytes=48 * 1024 * 1024,
        ),
        name="sinkhorn_fused",
    )(eps_sched, xy, xyt)
    return out[0, 0, 0] - out[1, 0, 0]


# speedup vs baseline: 1.5342x; 1.3347x over previous
import functools
import math

import jax
import jax.numpy as jnp
from jax import lax
from jax.experimental import pallas as pl
from jax.experimental.pallas import tpu as pltpu

_LOG2E = 1.4426950408889634
_LN2 = 0.6931471805599453


def _sink_body(eps_ref, xy_ref, xyt_ref, out_ref, pots_ref, bc_ref, hk_ref,
               mstat_ref, sstat_ref, *, R, NB, nlog):
    p = pl.program_id(0)
    t = pl.program_id(1)
    s = pl.program_id(2)
    T = pl.num_programs(1)
    N = NB * R

    eps = eps_ref[t]
    inv_eps = 1.0 / eps
    alpha2 = (-0.5 * _LOG2E) * inv_eps
    g2 = _LOG2E * inv_eps
    col_sel = jnp.where(p == 0, 1 - s, s)
    rb = lax.rem(t, 2)
    wb = 1 - rb

    is_first = t == 0
    mid = jnp.logical_and(t >= 1, t <= T - 2)
    hbase = nlog * _LOG2E

    for c in range(NB):
        colsk = xy_ref[col_sel, c * R:(c + 1) * R, :]
        csqk = jnp.sum(colsk * colsk, axis=1, keepdims=True)
        bc_ref[c * R:(c + 1) * R, :] = jnp.broadcast_to(
            alpha2 * csqk, (R, 128))
        pb = pots_ref[col_sel, rb, c]
        hrow = jnp.where(is_first, hbase, hbase + g2 * pb)
        hk_ref[c * R:(c + 1) * R, :] = jnp.broadcast_to(
            jnp.transpose(hrow), (R, 128))

    rep = R // 128

    def row_block(i, carry):
        rowsT = xyt_ref[s, :, pl.ds(i * R, R)]
        br2 = alpha2 * jnp.sum(rowsT * rowsT, axis=0, keepdims=True)

        for c in range(NB):
            colsk = xy_ref[col_sel, c * R:(c + 1) * R, :]
            dk = jnp.dot(colsk, rowsT, preferred_element_type=jnp.float32)
            bck = pltpu.repeat(bc_ref[c * R:(c + 1) * R, :], rep, axis=1)
            z = dk * g2 + br2 + bck
            hkk = pltpu.repeat(hk_ref[c * R:(c + 1) * R, :], rep, axis=1)
            sc2 = jnp.minimum(z, 0.0) + hkk
            mc = jnp.max(sc2, axis=0, keepdims=True)
            mstat_ref[c] = mc
            sstat_ref[c] = jnp.sum(jnp.exp2(sc2 - mc), axis=0,
                                   keepdims=True)

        ms = mstat_ref[:, 0, :]
        ss = sstat_ref[:, 0, :]
        M = jnp.max(ms, axis=0, keepdims=True)
        S = jnp.sum(ss * jnp.exp2(ms - M), axis=0, keepdims=True)

        smin = (-_LN2 * eps) * (M + jnp.log2(S))
        old = pots_ref[s, rb, i]
        new = jnp.where(mid, 0.5 * (old + smin), smin)
        pots_ref[s, wb, i] = new

        @pl.when(t == T - 1)
        def _():
            contrib = jnp.sum(new, axis=1, keepdims=True) * (1.0 / N)
            c3 = contrib.reshape(1, 1, 1)
            first = jnp.logical_and(s == 0, i == 0)
            out_ref[...] = jnp.where(first, c3, out_ref[...] + c3)

        return carry

    lax.fori_loop(0, NB, row_block, 0)


def kernel(g, Y, eps_list):
    N, D = g.shape
    M = Y.shape[0]
    assert N == M, "kernel assumes equal-sized point clouds"
    R = 256 if N % 256 == 0 else 128
    NB = N // R
    nlog = -math.log(float(N))

    xy = jnp.stack([g, Y])
    xyt = jnp.stack([g.T, Y.T])
    eps_sched = jnp.concatenate([eps_list[:1], eps_list, eps_list[-1:]])
    T = eps_sched.shape[0]

    body = functools.partial(_sink_body, R=R, NB=NB, nlog=nlog)
    out = pl.pallas_call(
        body,
        grid=(2, T, 2),
        in_specs=[
            pl.BlockSpec(memory_space=pltpu.SMEM),
            pl.BlockSpec((2, N, D), lambda p, t, s: (0, 0, 0)),
            pl.BlockSpec((2, D, N), lambda p, t, s: (0, 0, 0)),
        ],
        out_specs=pl.BlockSpec((1, 1, 1), lambda p, t, s: (p, 0, 0)),
        out_shape=jax.ShapeDtypeStruct((2, 1, 1), jnp.float32),
        scratch_shapes=[
            pltpu.VMEM((2, 2, NB, 1, R), jnp.float32),
            pltpu.VMEM((N, 128), jnp.float32),
            pltpu.VMEM((N, 128), jnp.float32),
            pltpu.VMEM((NB, 1, R), jnp.float32),
            pltpu.VMEM((NB, 1, R), jnp.float32),
        ],
        compiler_params=pltpu.CompilerParams(
            dimension_semantics=("arbitrary", "arbitrary", "arbitrary"),
            vmem_limit_bytes=48 * 1024 * 1024,
        ),
        name="sinkhorn_fused",
    )(eps_sched, xy, xyt)
    return out[0, 0, 0] - out[1, 0, 0]


# speedup vs baseline: 2.0609x; 1.3434x over previous
import functools
import math

import jax
import jax.numpy as jnp
from jax import lax
from jax.experimental import pallas as pl
from jax.experimental.pallas import tpu as pltpu

_LOG2E = 1.4426950408889634
_LN2 = 0.6931471805599453


def _sink_body(eps_ref, xy_ref, xyt_ref, out_ref, pots_ref, hb_ref,
               mstat_ref, sstat_ref, *, R, NB, CW, NC, nlog):
    p = pl.program_id(0)
    t = pl.program_id(1)
    s = pl.program_id(2)
    T = pl.num_programs(1)
    N = NB * R

    eps = eps_ref[t]
    inv_eps = 1.0 / eps
    alpha2 = (-0.5 * _LOG2E) * inv_eps
    g2 = _LOG2E * inv_eps
    col_sel = jnp.where(p == 0, 1 - s, s)
    rb = lax.rem(t, 2)
    wb = 1 - rb

    is_first = t == 0
    mid = jnp.logical_and(t >= 1, t <= T - 2)
    hbase = nlog * _LOG2E

    for c in range(NB):
        colsk = xy_ref[col_sel, c * R:(c + 1) * R, :]
        csqk = jnp.sum(colsk * colsk, axis=1, keepdims=True)
        pb = pots_ref[col_sel, rb, c]
        hrow = jnp.where(is_first, hbase, hbase + g2 * pb)
        hb_ref[c * R:(c + 1) * R, :] = jnp.broadcast_to(
            alpha2 * csqk + jnp.transpose(hrow), (R, 128))

    rep = R // 128

    def one_block(i, koff):
        rowsT = xyt_ref[s, :, pl.ds(i * R, R)]
        br2 = alpha2 * jnp.sum(rowsT * rowsT, axis=0, keepdims=True)

        for c in range(NC):
            colsk = xy_ref[col_sel, c * CW:(c + 1) * CW, :]
            dk = jnp.dot(colsk, rowsT, preferred_element_type=jnp.float32)
            hbk = pltpu.repeat(hb_ref[c * CW:(c + 1) * CW, :], rep, axis=1)
            sc2 = dk * g2 + hbk
            mc = jnp.max(sc2, axis=0, keepdims=True)
            mstat_ref[koff + c] = mc
            sstat_ref[koff + c] = jnp.sum(jnp.exp2(sc2 - mc), axis=0,
                                          keepdims=True)

        ms = mstat_ref[koff:koff + NC, 0, :]
        ss = sstat_ref[koff:koff + NC, 0, :]
        M = jnp.max(ms, axis=0, keepdims=True)
        S = jnp.sum(ss * jnp.exp2(ms - M), axis=0, keepdims=True)

        smin = (-_LN2 * eps) * (br2 + M + jnp.log2(S))
        old = pots_ref[s, rb, i]
        new = jnp.where(mid, 0.5 * (old + smin), smin)
        pots_ref[s, wb, i] = new

        @pl.when(t == T - 1)
        def _():
            contrib = jnp.sum(new, axis=1, keepdims=True) * (1.0 / N)
            c3 = contrib.reshape(1, 1, 1)
            first = jnp.logical_and(s == 0, i == 0)
            out_ref[...] = jnp.where(first, c3, out_ref[...] + c3)

    def row_block(i, carry):
        one_block(i, 0)
        return carry

    lax.fori_loop(0, NB, row_block, 0)


def kernel(g, Y, eps_list):
    N, D = g.shape
    M = Y.shape[0]
    assert N == M, "kernel assumes equal-sized point clouds"
    R = 256 if N % 256 == 0 else 128
    NB = N // R
    CW = 512
    NC = N // CW
    nlog = -math.log(float(N))

    xy = jnp.stack([g, Y])
    xyt = jnp.stack([g.T, Y.T])
    eps_sched = jnp.concatenate([eps_list[:1], eps_list, eps_list[-1:]])
    T = eps_sched.shape[0]

    body = functools.partial(_sink_body, R=R, NB=NB, CW=CW, NC=NC, nlog=nlog)
    out = pl.pallas_call(
        body,
        grid=(2, T, 2),
        in_specs=[
            pl.BlockSpec(memory_space=pltpu.SMEM),
            pl.BlockSpec((2, N, D), lambda p, t, s: (0, 0, 0)),
            pl.BlockSpec((2, D, N), lambda p, t, s: (0, 0, 0)),
        ],
        out_specs=pl.BlockSpec((1, 1, 1), lambda p, t, s: (p, 0, 0)),
        out_shape=jax.ShapeDtypeStruct((2, 1, 1), jnp.float32),
        scratch_shapes=[
            pltpu.VMEM((2, 2, NB, 1, R), jnp.float32),
            pltpu.VMEM((N, 128), jnp.float32),
            pltpu.VMEM((NC, 1, R), jnp.float32),
            pltpu.VMEM((NC, 1, R), jnp.float32),
        ],
        compiler_params=pltpu.CompilerParams(
            dimension_semantics=("arbitrary", "arbitrary", "arbitrary"),
            vmem_limit_bytes=48 * 1024 * 1024,
        ),
        name="sinkhorn_fused",
    )(eps_sched, xy, xyt)
    return out[0, 0, 0] - out[1, 0, 0]
